# Initial kernel scaffold; baseline (speedup 1.0000x reference)
import jax
import jax.numpy as jnp
from jax.experimental import pallas as pl


def kernel(node_features, node_preprocess_feat, edge_index, edge_features, node_mask, edge_mask, W_enc, b_enc, W_edge, b_edge, W1, b1, W2, b2, eps, gamma, beta, Vw1, Vb1, Vw2, Vb2, W_pred, b_pred):
    raise NotImplementedError("write your pallas kernel here")


# speedup vs baseline: 1.4330x; 1.4330x over previous
import functools

import jax
import jax.numpy as jnp
from jax import lax
from jax.experimental import pallas as pl
from jax.experimental.pallas import tpu as pltpu
from jax.experimental.pallas import tpu_sc as plsc

BS, N, E, FIN, FE, D, L, T = 128, 196, 3136, 9, 3, 128, 3, 12
NTOT = BS * N
ETOT = BS * E
F32 = jnp.float32

NTILES = 32
GPT = BS // NTILES
EPT = GPT * E
RPT = GPT * N
KBLK = 64
NB = EPT // KBLK


def _bdot(a, b, dims=((1,), (0,))):
    return jax.lax.dot_general(a.astype(jnp.bfloat16), b.astype(jnp.bfloat16),
                               (dims, ((), ())), preferred_element_type=F32)


def _enc_body(x_ref, w_ref, b_ref, o_ref):
    o_ref[...] = _bdot(x_ref[...], w_ref[...]) + b_ref[...]


def _eemb_body(ea_ref, we_ref, be_ref, o_ref):
    o_ref[0] = _bdot(ea_ref[0], we_ref[...]) + be_ref[...]


def _prep_body(ei_ref, srcg_ref, dstt_ref):
    b = pl.program_id(0)
    srcg_ref[0] = ei_ref[0, :, 0:1] + b * N
    dstt_ref[0] = ei_ref[0, :, 1:2] + (b % GPT) * N


def _hplus_body(h_ref, vn_ref, o_ref):
    o_ref[0] = h_ref[0] + vn_ref[0]


def _mlp_body(hp_ref, agg_ref, w1_ref, b1_ref, w2_ref, b2_ref, eps_ref,
              u_ref, ps_ref, pq_ref):
    z = (1.0 + eps_ref[0, 0]) * hp_ref[0] + agg_ref[0]
    t = jnp.maximum(_bdot(z, w1_ref[...]) + b1_ref[...], 0.0)
    u = _bdot(t, w2_ref[...]) + b2_ref[...]
    u_ref[0] = u
    ps_ref[0, 0] = jnp.sum(u, axis=0)
    pq_ref[0, 0] = jnp.sum(u * u, axis=0)


def _bn_body(u_ref, ps_ref, pq_ref, g_ref, bt_ref, o_ref, s_ref, *, do_relu):
    tot = jnp.sum(ps_ref[...], axis=(0, 1))
    totq = jnp.sum(pq_ref[...], axis=(0, 1))
    mean = (tot / NTOT)[None, :]
    var = (totq / NTOT)[None, :] - mean * mean
    out = (u_ref[0] - mean) / jnp.sqrt(var + 1e-5) * g_ref[...] + bt_ref[...]
    if do_relu:
        out = jnp.maximum(out, 0.0)
    o_ref[0] = out
    s_ref[0, 0] = jnp.sum(out, axis=0)


def _vn_body(s_ref, vn_ref, w1_ref, b1_ref, w2_ref, b2_ref, o_ref):
    vt = s_ref[:, 0, :] + vn_ref[:, 0, :]
    t = jnp.maximum(_bdot(vt, w1_ref[...]) + b1_ref[...], 0.0)
    o_ref[...] = (_bdot(t, w2_ref[...]) + b2_ref[...])[:, None, :]


def _pred_body(s_ref, w_ref, b_ref, o_ref):
    hg = s_ref[:, 0, :] * (1.0 / N)
    o_ref[...] = _bdot(hg, w_ref[...]) + b_ref[...]


def _sc_body(h_hbm, src_hbm, dst_hbm, emb_hbm, agg_hbm,
             idx_v, dst_v, rows_v, emb_v, agg_v, sem):
    c = lax.axis_index("c")
    s = lax.axis_index("s")
    t = c * 16 + s

    zero16 = jnp.zeros((16,), F32)

    def zrow(r, carry):
        for q in range(8):
            agg_v[r, pl.ds(16 * q, 16)] = zero16
        return carry

    lax.fori_loop(0, RPT, zrow, 0)

    ebase = t * EPT

    def blk(i, carry):
        base = ebase + i * KBLK
        pltpu.sync_copy(src_hbm.at[pl.ds(base, KBLK)], idx_v)
        pltpu.sync_copy(dst_hbm.at[pl.ds(base, KBLK)], dst_v.at[pl.ds(0, KBLK)])
        pltpu.sync_copy(emb_hbm.at[pl.ds(base, KBLK)], emb_v)
        pltpu.async_copy(h_hbm.at[idx_v], rows_v, sem).wait()

        def edge(e, carry2):
            d = dst_v[pl.ds(e, 16)][0]
            for q in range(8):
                hv = rows_v[e, pl.ds(16 * q, 16)]
                ev = emb_v[e, pl.ds(16 * q, 16)]
                m = jnp.maximum(hv + ev, 0.0)
                agg_v[d, pl.ds(16 * q, 16)] = agg_v[d, pl.ds(16 * q, 16)] + m
            return carry2

        lax.fori_loop(0, KBLK, edge, 0)
        return carry

    lax.fori_loop(0, NB, blk, 0)
    pltpu.sync_copy(agg_v, agg_hbm.at[pl.ds(t * RPT, RPT)])


def _sc_msgpass(hplus, src_g, dst_t, eemb):
    mesh = plsc.VectorSubcoreMesh(
        core_axis_name="c", subcore_axis_name="s", num_cores=2, num_subcores=16
    )
    return pl.kernel(
        _sc_body,
        out_type=jax.ShapeDtypeStruct((NTOT, D), F32),
        mesh=mesh,
        scratch_types=[
            pltpu.VMEM((KBLK,), jnp.int32),
            pltpu.VMEM((KBLK + 16,), jnp.int32),
            pltpu.VMEM((KBLK, D), F32),
            pltpu.VMEM((KBLK, D), F32),
            pltpu.VMEM((RPT, D), F32),
            pltpu.SemaphoreType.DMA,
        ],
    )(hplus, src_g, dst_t, eemb)


def kernel(node_features, node_preprocess_feat, edge_index, edge_features,
           node_mask, edge_mask, W_enc, b_enc, W_edge, b_edge, W1, b1, W2, b2,
           eps, gamma, beta, Vw1, Vb1, Vw2, Vb2, W_pred, b_pred):
    x2d = node_features.reshape(NTOT, FIN)
    ei = edge_index.astype(jnp.int32)

    whole = lambda *shape: pl.BlockSpec(shape, lambda b: (0,) * len(shape))
    per_graph = lambda *rest: pl.BlockSpec(
        (1,) + rest, lambda b: (b,) + (0,) * len(rest))

    h = pl.pallas_call(
        _enc_body,
        grid=(8,),
        in_specs=[
            pl.BlockSpec((NTOT // 8, FIN), lambda i: (i, 0)),
            pl.BlockSpec((FIN, D), lambda i: (0, 0)),
            pl.BlockSpec((1, D), lambda i: (0, 0)),
        ],
        out_specs=pl.BlockSpec((NTOT // 8, D), lambda i: (i, 0)),
        out_shape=jax.ShapeDtypeStruct((NTOT, D), F32),
    )(x2d, W_enc, b_enc.reshape(1, D))

    src_g, dst_t = pl.pallas_call(
        _prep_body,
        grid=(BS,),
        in_specs=[per_graph(E, 2)],
        out_specs=[per_graph(E, 1), per_graph(E, 1)],
        out_shape=[
            jax.ShapeDtypeStruct((BS, E, 1), jnp.int32),
            jax.ShapeDtypeStruct((BS, E, 1), jnp.int32),
        ],
    )(ei)
    src_g = src_g.reshape(ETOT)
    dst_t = dst_t.reshape(ETOT)

    eemb_call = pl.pallas_call(
        _eemb_body,
        grid=(BS,),
        in_specs=[per_graph(E, FE), whole(FE, D), whole(1, D)],
        out_specs=per_graph(E, D),
        out_shape=jax.ShapeDtypeStruct((BS, E, D), F32),
    )

    hplus_call = pl.pallas_call(
        _hplus_body,
        grid=(BS,),
        in_specs=[per_graph(N, D), per_graph(1, D)],
        out_specs=per_graph(N, D),
        out_shape=jax.ShapeDtypeStruct((BS, N, D), F32),
    )

    mlp_call = pl.pallas_call(
        _mlp_body,
        grid=(BS,),
        in_specs=[
            per_graph(N, D),
            per_graph(N, D),
            whole(D, 2 * D),
            whole(1, 2 * D),
            whole(2 * D, D),
            whole(1, D),
            whole(1, 1),
        ],
        out_specs=[per_graph(N, D), per_graph(1, D), per_graph(1, D)],
        out_shape=[
            jax.ShapeDtypeStruct((BS, N, D), F32),
            jax.ShapeDtypeStruct((BS, 1, D), F32),
            jax.ShapeDtypeStruct((BS, 1, D), F32),
        ],
    )

    def bn_call(do_relu):
        return pl.pallas_call(
            functools.partial(_bn_body, do_relu=do_relu),
            grid=(BS,),
            in_specs=[
                per_graph(N, D),
                whole(BS, 1, D),
                whole(BS, 1, D),
                whole(1, D),
                whole(1, D),
            ],
            out_specs=[per_graph(N, D), per_graph(1, D)],
            out_shape=[
                jax.ShapeDtypeStruct((BS, N, D), F32),
                jax.ShapeDtypeStruct((BS, 1, D), F32),
            ],
        )

    vn_call = pl.pallas_call(
        _vn_body,
        out_shape=jax.ShapeDtypeStruct((BS, 1, D), F32),
    )

    vn = None
    hp3 = h.reshape(BS, N, D)
    for l in range(L):
        if l > 0:
            hp3 = hplus_call(hp3, vn)
        hplus = hp3.reshape(NTOT, D)
        eemb = eemb_call(edge_features, W_edge[l], b_edge[l].reshape(1, D))
        agg = _sc_msgpass(hplus, src_g, dst_t, eemb.reshape(ETOT, D))
        u, ps, pq = mlp_call(
            hplus.reshape(BS, N, D), agg.reshape(BS, N, D),
            W1[l], b1[l].reshape(1, 2 * D), W2[l], b2[l].reshape(1, D),
            eps[l].reshape(1, 1),
        )
        hp3, s = bn_call(do_relu=(l < L - 1))(
            u, ps, pq, gamma[l].reshape(1, D), beta[l].reshape(1, D)
        )
        if l < L - 1:
            vn_prev = vn if vn is not None else jnp.zeros((BS, 1, D), F32)
            vn = vn_call(
                s, vn_prev, Vw1[l], Vb1[l].reshape(1, 2 * D), Vw2[l],
                Vb2[l].reshape(1, D),
            )

    out = pl.pallas_call(
        _pred_body,
        out_shape=jax.ShapeDtypeStruct((BS, T), F32),
    )(s, W_pred, b_pred.reshape(1, T))
    return out


# speedup vs baseline: 1.7606x; 1.2286x over previous
import functools

import jax
import jax.numpy as jnp
from jax import lax
from jax.experimental import pallas as pl
from jax.experimental.pallas import tpu as pltpu
from jax.experimental.pallas import tpu_sc as plsc

BS, N, E, FIN, FE, D, L, T = 128, 196, 3136, 9, 3, 128, 3, 12
NTOT = BS * N
ETOT = BS * E
F32 = jnp.float32

NTILES = 32
GPT = BS // NTILES
EPT = GPT * E
RPT = GPT * N
KBLK = 112
NB = EPT // KBLK


def _bdot(a, b, dims=((1,), (0,))):
    return jax.lax.dot_general(a.astype(jnp.bfloat16), b.astype(jnp.bfloat16),
                               (dims, ((), ())), preferred_element_type=F32)


def _enc_body(x_ref, w_ref, b_ref, o_ref):
    o_ref[...] = _bdot(x_ref[...], w_ref[...]) + b_ref[...]


def _eemb_body(ea_ref, we_ref, be_ref, o_ref):
    o_ref[0] = _bdot(ea_ref[0], we_ref[...]) + be_ref[...]


def _prep_body(ei_ref, srcg_ref, dstt_ref):
    b = pl.program_id(0)
    srcg_ref[0] = ei_ref[0, :, 0:1] + b * N
    dstt_ref[0] = ei_ref[0, :, 1:2] + (b % GPT) * N


def _hplus_body(h_ref, vn_ref, o_ref):
    o_ref[0] = h_ref[0] + vn_ref[0]


def _mlp_body(hp_ref, agg_ref, w1_ref, b1_ref, w2_ref, b2_ref, eps_ref,
              u_ref, ps_ref, pq_ref):
    z = (1.0 + eps_ref[0, 0]) * hp_ref[0] + agg_ref[0]
    t = jnp.maximum(_bdot(z, w1_ref[...]) + b1_ref[...], 0.0)
    u = _bdot(t, w2_ref[...]) + b2_ref[...]
    u_ref[0] = u
    ps_ref[0, 0] = jnp.sum(u, axis=0)
    pq_ref[0, 0] = jnp.sum(u * u, axis=0)


def _bn_body(u_ref, ps_ref, pq_ref, g_ref, bt_ref, o_ref, s_ref, *, do_relu):
    tot = jnp.sum(ps_ref[...], axis=(0, 1))
    totq = jnp.sum(pq_ref[...], axis=(0, 1))
    mean = (tot / NTOT)[None, :]
    var = (totq / NTOT)[None, :] - mean * mean
    out = (u_ref[0] - mean) / jnp.sqrt(var + 1e-5) * g_ref[...] + bt_ref[...]
    if do_relu:
        out = jnp.maximum(out, 0.0)
    o_ref[0] = out
    s_ref[0, 0] = jnp.sum(out, axis=0)


def _vn_body(s_ref, vn_ref, w1_ref, b1_ref, w2_ref, b2_ref, o_ref):
    vt = s_ref[:, 0, :] + vn_ref[:, 0, :]
    t = jnp.maximum(_bdot(vt, w1_ref[...]) + b1_ref[...], 0.0)
    o_ref[...] = (_bdot(t, w2_ref[...]) + b2_ref[...])[:, None, :]


def _pred_body(s_ref, w_ref, b_ref, o_ref):
    hg = s_ref[:, 0, :] * (1.0 / N)
    o_ref[...] = _bdot(hg, w_ref[...]) + b_ref[...]


def _sc_body(h_hbm, src_hbm, dst_hbm, emb_hbm, agg_hbm,
             idx_v, dst_v, rows_v, emb_v, agg_v, sem_i, sem_d, sem_e, sem_g):
    c = lax.axis_index("c")
    s = lax.axis_index("s")
    t = c * 16 + s

    zero16 = jnp.zeros((16,), F32)

    def zrow(r, carry):
        for q in range(8):
            agg_v[r, pl.ds(16 * q, 16)] = zero16
        return carry

    lax.fori_loop(0, RPT, zrow, 0)

    ebase = t * EPT

    def blk(i, carry):
        base = ebase + i * KBLK
        cp_i = pltpu.async_copy(src_hbm.at[pl.ds(base, KBLK)], idx_v, sem_i)
        cp_d = pltpu.async_copy(dst_hbm.at[pl.ds(base, KBLK)],
                                dst_v.at[pl.ds(0, KBLK)], sem_d)
        cp_e = pltpu.async_copy(emb_hbm.at[pl.ds(base, KBLK)], emb_v, sem_e)
        cp_i.wait()
        cp_g = pltpu.async_copy(h_hbm.at[idx_v], rows_v, sem_g)
        cp_d.wait()
        cp_e.wait()
        cp_g.wait()

        def edge(e, carry2):
            d = dst_v[pl.ds(e, 16)][0]
            for q in range(8):
                hv = rows_v[e, pl.ds(16 * q, 16)]
                ev = emb_v[e, pl.ds(16 * q, 16)]
                m = jnp.maximum(hv + ev, 0.0)
                agg_v[d, pl.ds(16 * q, 16)] = agg_v[d, pl.ds(16 * q, 16)] + m
            return carry2

        lax.fori_loop(0, KBLK, edge, 0)
        return carry

    lax.fori_loop(0, NB, blk, 0)
    pltpu.sync_copy(agg_v, agg_hbm.at[pl.ds(t * RPT, RPT)])


def _sc_msgpass(hplus, src_g, dst_t, eemb):
    mesh = plsc.VectorSubcoreMesh(
        core_axis_name="c", subcore_axis_name="s", num_cores=2, num_subcores=16
    )
    return pl.kernel(
        _sc_body,
        out_type=jax.ShapeDtypeStruct((NTOT, D), F32),
        mesh=mesh,
        scratch_types=[
            pltpu.VMEM((KBLK,), jnp.int32),
            pltpu.VMEM((KBLK + 16,), jnp.int32),
            pltpu.VMEM((KBLK, D), F32),
            pltpu.VMEM((KBLK, D), F32),
            pltpu.VMEM((RPT, D), F32),
            pltpu.SemaphoreType.DMA,
            pltpu.SemaphoreType.DMA,
            pltpu.SemaphoreType.DMA,
            pltpu.SemaphoreType.DMA,
        ],
    )(hplus, src_g, dst_t, eemb)


def kernel(node_features, node_preprocess_feat, edge_index, edge_features,
           node_mask, edge_mask, W_enc, b_enc, W_edge, b_edge, W1, b1, W2, b2,
           eps, gamma, beta, Vw1, Vb1, Vw2, Vb2, W_pred, b_pred):
    x2d = node_features.reshape(NTOT, FIN)
    ei = edge_index.astype(jnp.int32)

    whole = lambda *shape: pl.BlockSpec(shape, lambda b: (0,) * len(shape))
    per_graph = lambda *rest: pl.BlockSpec(
        (1,) + rest, lambda b: (b,) + (0,) * len(rest))

    h = pl.pallas_call(
        _enc_body,
        grid=(8,),
        in_specs=[
            pl.BlockSpec((NTOT // 8, FIN), lambda i: (i, 0)),
            pl.BlockSpec((FIN, D), lambda i: (0, 0)),
            pl.BlockSpec((1, D), lambda i: (0, 0)),
        ],
        out_specs=pl.BlockSpec((NTOT // 8, D), lambda i: (i, 0)),
        out_shape=jax.ShapeDtypeStruct((NTOT, D), F32),
    )(x2d, W_enc, b_enc.reshape(1, D))

    src_g, dst_t = pl.pallas_call(
        _prep_body,
        grid=(BS,),
        in_specs=[per_graph(E, 2)],
        out_specs=[per_graph(E, 1), per_graph(E, 1)],
        out_shape=[
            jax.ShapeDtypeStruct((BS, E, 1), jnp.int32),
            jax.ShapeDtypeStruct((BS, E, 1), jnp.int32),
        ],
    )(ei)
    src_g = src_g.reshape(ETOT)
    dst_t = dst_t.reshape(ETOT)

    eemb_call = pl.pallas_call(
        _eemb_body,
        grid=(BS,),
        in_specs=[per_graph(E, FE), whole(FE, D), whole(1, D)],
        out_specs=per_graph(E, D),
        out_shape=jax.ShapeDtypeStruct((BS, E, D), F32),
    )

    hplus_call = pl.pallas_call(
        _hplus_body,
        grid=(BS,),
        in_specs=[per_graph(N, D), per_graph(1, D)],
        out_specs=per_graph(N, D),
        out_shape=jax.ShapeDtypeStruct((BS, N, D), F32),
    )

    mlp_call = pl.pallas_call(
        _mlp_body,
        grid=(BS,),
        in_specs=[
            per_graph(N, D),
            per_graph(N, D),
            whole(D, 2 * D),
            whole(1, 2 * D),
            whole(2 * D, D),
            whole(1, D),
            whole(1, 1),
        ],
        out_specs=[per_graph(N, D), per_graph(1, D), per_graph(1, D)],
        out_shape=[
            jax.ShapeDtypeStruct((BS, N, D), F32),
            jax.ShapeDtypeStruct((BS, 1, D), F32),
            jax.ShapeDtypeStruct((BS, 1, D), F32),
        ],
    )

    def bn_call(do_relu):
        return pl.pallas_call(
            functools.partial(_bn_body, do_relu=do_relu),
            grid=(BS,),
            in_specs=[
                per_graph(N, D),
                whole(BS, 1, D),
                whole(BS, 1, D),
                whole(1, D),
                whole(1, D),
            ],
            out_specs=[per_graph(N, D), per_graph(1, D)],
            out_shape=[
                jax.ShapeDtypeStruct((BS, N, D), F32),
                jax.ShapeDtypeStruct((BS, 1, D), F32),
            ],
        )

    vn_call = pl.pallas_call(
        _vn_body,
        out_shape=jax.ShapeDtypeStruct((BS, 1, D), F32),
    )

    vn = None
    hp3 = h.reshape(BS, N, D)
    for l in range(L):
        if l > 0:
            hp3 = hplus_call(hp3, vn)
        hplus = hp3.reshape(NTOT, D)
        eemb = eemb_call(edge_features, W_edge[l], b_edge[l].reshape(1, D))
        agg = _sc_msgpass(hplus, src_g, dst_t, eemb.reshape(ETOT, D))
        u, ps, pq = mlp_call(
            hplus.reshape(BS, N, D), agg.reshape(BS, N, D),
            W1[l], b1[l].reshape(1, 2 * D), W2[l], b2[l].reshape(1, D),
            eps[l].reshape(1, 1),
        )
        hp3, s = bn_call(do_relu=(l < L - 1))(
            u, ps, pq, gamma[l].reshape(1, D), beta[l].reshape(1, D)
        )
        if l < L - 1:
            vn_prev = vn if vn is not None else jnp.zeros((BS, 1, D), F32)
            vn = vn_call(
                s, vn_prev, Vw1[l], Vb1[l].reshape(1, 2 * D), Vw2[l],
                Vb2[l].reshape(1, D),
            )

    out = pl.pallas_call(
        _pred_body,
        out_shape=jax.ShapeDtypeStruct((BS, T), F32),
    )(s, W_pred, b_pred.reshape(1, T))
    return out
